# baseline (speedup 1.0000x reference)
import jax
import jax.numpy as jnp
from jax import lax
from jax.experimental import pallas as pl
from jax.experimental.pallas import tpu as pltpu

D = 4096
E = 64
T = 8192
BT = 512


def _body(x_ref, w_ref, oh_ref, tp_ref, lg_ref):
    lg = lax.dot_general(
        x_ref[...], w_ref[...],
        (((1,), (1,)), ((), ())),
        preferred_element_type=jnp.float32,
    )
    lg_ref[...] = lg
    m = jnp.max(lg, axis=1, keepdims=True)
    iota = lax.broadcasted_iota(jnp.int32, (BT, E), 1)
    am = jnp.min(jnp.where(lg == m, iota, E), axis=1, keepdims=True)
    oh_ref[...] = (iota == am).astype(jnp.int32)
    tp_ref[...] = 1.0 / jnp.sum(jnp.exp(lg - m), axis=1, keepdims=True)


def kernel(x, W):
    oh, tp, lg = pl.pallas_call(
        _body,
        grid=(T // BT,),
        in_specs=[
            pl.BlockSpec((BT, D), lambda i: (i, 0)),
            pl.BlockSpec((E, D), lambda i: (0, 0)),
        ],
        out_specs=(
            pl.BlockSpec((BT, E), lambda i: (i, 0)),
            pl.BlockSpec((BT, 1), lambda i: (i, 0)),
            pl.BlockSpec((BT, E), lambda i: (i, 0)),
        ),
        out_shape=(
            jax.ShapeDtypeStruct((T, E), jnp.int32),
            jax.ShapeDtypeStruct((T, 1), jnp.float32),
            jax.ShapeDtypeStruct((T, E), jnp.float32),
        ),
        compiler_params=pltpu.CompilerParams(
            dimension_semantics=("parallel",),
        ),
    )(x, W)
    return oh, tp, lg


# speedup vs baseline: 1.0068x; 1.0068x over previous
import jax
import jax.numpy as jnp
from jax import lax
from jax.experimental import pallas as pl
from jax.experimental.pallas import tpu as pltpu

D = 4096
E = 64
T = 8192
BT = 512
H = D // 2


def _body(x1_ref, x2_ref, w_ref, oh_ref, tp_ref, lg_ref):
    lg = lax.dot_general(
        x1_ref[...], w_ref[:, :H],
        (((1,), (1,)), ((), ())),
        preferred_element_type=jnp.float32,
    ) + lax.dot_general(
        x2_ref[...], w_ref[:, H:],
        (((1,), (1,)), ((), ())),
        preferred_element_type=jnp.float32,
    )
    lg_ref[...] = lg
    m = jnp.max(lg, axis=1, keepdims=True)
    iota = lax.broadcasted_iota(jnp.int32, (BT, E), 1)
    am = jnp.min(jnp.where(lg == m, iota, E), axis=1, keepdims=True)
    oh_ref[...] = (iota == am).astype(jnp.int32)
    tp_ref[...] = 1.0 / jnp.sum(jnp.exp(lg - m), axis=1, keepdims=True)


def kernel(x, W):
    oh, tp, lg = pl.pallas_call(
        _body,
        grid=(T // BT,),
        in_specs=[
            pl.BlockSpec((BT, H), lambda i: (i, 0)),
            pl.BlockSpec((BT, H), lambda i: (i, 1)),
            pl.BlockSpec((E, D), lambda i: (0, 0)),
        ],
        out_specs=(
            pl.BlockSpec((BT, E), lambda i: (i, 0)),
            pl.BlockSpec((BT, 1), lambda i: (i, 0)),
            pl.BlockSpec((BT, E), lambda i: (i, 0)),
        ),
        out_shape=(
            jax.ShapeDtypeStruct((T, E), jnp.int32),
            jax.ShapeDtypeStruct((T, 1), jnp.float32),
            jax.ShapeDtypeStruct((T, E), jnp.float32),
        ),
        compiler_params=pltpu.CompilerParams(
            dimension_semantics=("arbitrary",),
        ),
    )(x, x, W)
    return oh, tp, lg
